# baseline (speedup 1.0000x reference)
import functools
import math

import jax
import jax.numpy as jnp
from jax import lax
from jax.experimental import pallas as pl
from jax.experimental.pallas import tpu as pltpu
from jax.experimental.pallas import tpu_sc as plsc

N = 10000
E = 160000
F = 256
NP = 10240
NR = NP // 128
EP = 163840
EB = EP // 128
DUMMY = 10008
K1 = math.ceil(0.5 * N)
K2 = math.ceil(0.5 * K1)
K3 = math.ceil(0.5 * K2)

BW = 64
NB_SUB = EP // 16 // BW
CHK = 32
NB_TILE = EB // 32
ROWS_SUB = NP // 16

_mesh = plsc.VectorSubcoreMesh(core_axis_name="c", subcore_axis_name="s")
_f32 = jnp.float32
_i32 = jnp.int32


@functools.partial(
    pl.kernel,
    mesh=_mesh,
    compiler_params=pltpu.CompilerParams(needs_layout_passes=False),
    out_type=jax.ShapeDtypeStruct((2, NP, 128), _f32),
    scratch_types=[
        pltpu.VMEM((CHK, BW), _i32),
        pltpu.VMEM((CHK, BW), _i32),
        pltpu.VMEM((BW, 128), _f32),
        pltpu.VMEM((BW, 128), _f32),
        pltpu.VMEM_SHARED((NP, 128), _f32),
        pltpu.SemaphoreType.DMA,
        pltpu.SemaphoreType.DMA,
    ],
)
def _sc_agg(xws0, xws1, row3d, col3d, zeros2d, out,
            idxr, idxc, gbuf0, gbuf1, acc, sem0, sem1):
    c = lax.axis_index("c")
    s = lax.axis_index("s")
    pltpu.sync_copy(zeros2d, gbuf0)
    for j in range(ROWS_SUB // BW):
        pltpu.sync_copy(gbuf0, acc.at[pl.ds(s * ROWS_SUB + j * BW, BW)])
    plsc.subcore_barrier()

    def run(table):
        def chunk(q, carry):
            pltpu.sync_copy(row3d.at[s].at[pl.ds(q * CHK, CHK)], idxr)
            pltpu.sync_copy(col3d.at[s].at[pl.ds(q * CHK, CHK)], idxc)
            pltpu.async_copy(table.at[idxr.at[0]], gbuf0, sem0)

            def body(i, carry2):
                b0 = 2 * i
                b1 = 2 * i + 1
                pltpu.async_copy(table.at[idxr.at[b1]], gbuf1, sem1)
                pltpu.make_async_copy(table.at[idxr.at[b0]], gbuf0, sem0).wait()
                pltpu.sync_copy(gbuf0, acc.at[idxc.at[b0]], add=True)

                @pl.when(b0 + 2 < CHK)
                def _():
                    pltpu.async_copy(table.at[idxr.at[b0 + 2]], gbuf0, sem0)

                pltpu.make_async_copy(table.at[idxr.at[b1]], gbuf1, sem1).wait()
                pltpu.sync_copy(gbuf1, acc.at[idxc.at[b1]], add=True)
                return carry2
            lax.fori_loop(0, CHK // 2, body, 0)
            return carry
        lax.fori_loop(0, NB_SUB // CHK, chunk, 0)

    @pl.when(c == 0)
    def _():
        run(xws0)

    @pl.when(c == 1)
    def _():
        run(xws1)

    plsc.subcore_barrier()
    for j in range(ROWS_SUB // 128):
        pltpu.sync_copy(
            acc.at[pl.ds(s * ROWS_SUB + j * 128, 128)],
            out.at[c].at[pl.ds(s * ROWS_SUB + j * 128, 128)],
        )


EDGES_TILE = EP // 32


@functools.partial(
    pl.kernel,
    mesh=_mesh,
    compiler_params=pltpu.CompilerParams(needs_layout_passes=False),
    out_type=jax.ShapeDtypeStruct((32, NP), _f32),
    scratch_types=[
        pltpu.VMEM((EDGES_TILE,), _i32),
        pltpu.VMEM((EDGES_TILE,), _i32),
        pltpu.VMEM((NP,), _f32),
        pltpu.VMEM((NP,), _f32),
    ],
)
def _sc_sagg(row1d, col1d, xss, out, rowb, colb, tab, lacc):
    c = lax.axis_index("c")
    s = lax.axis_index("s")
    w = s * 2 + c
    pltpu.sync_copy(xss, tab)
    pltpu.sync_copy(row1d.at[pl.ds(w * EDGES_TILE, EDGES_TILE)], rowb)
    pltpu.sync_copy(col1d.at[pl.ds(w * EDGES_TILE, EDGES_TILE)], colb)

    def zero(i, carry):
        lacc[pl.ds(i * 16, 16)] = jnp.zeros((16,), _f32)
        return carry
    lax.fori_loop(0, NP // 16, zero, 0)

    def body(j, carry):
        rv = rowb[pl.ds(j * 16, 16)]
        cv = colb[pl.ds(j * 16, 16)]
        vals = plsc.load_gather(tab, [rv])
        plsc.addupdate_scatter(lacc, [cv], vals)
        return carry
    lax.fori_loop(0, EDGES_TILE // 16, body, 0)
    pltpu.sync_copy(lacc, out.at[w])


@functools.partial(
    pl.kernel,
    mesh=_mesh,
    compiler_params=pltpu.CompilerParams(needs_layout_passes=False),
    out_type=[
        jax.ShapeDtypeStruct((EP,), _i32),
        jax.ShapeDtypeStruct((32, NP), _f32),
    ],
    scratch_types=[
        pltpu.VMEM((EDGES_TILE,), _i32),
        pltpu.VMEM((EDGES_TILE,), _i32),
        pltpu.VMEM((EDGES_TILE,), _i32),
        pltpu.VMEM((NP,), _i32),
        pltpu.VMEM((NP,), _f32),
    ],
)
def _sc_edges(row1d, col1d, selpad, colout, degout, rowb, colb, colob, seltab, ldeg):
    c = lax.axis_index("c")
    s = lax.axis_index("s")
    w = s * 2 + c
    pltpu.sync_copy(selpad, seltab)
    pltpu.sync_copy(row1d.at[pl.ds(w * EDGES_TILE, EDGES_TILE)], rowb)
    pltpu.sync_copy(col1d.at[pl.ds(w * EDGES_TILE, EDGES_TILE)], colb)

    def zero(i, carry):
        ldeg[pl.ds(i * 16, 16)] = jnp.zeros((16,), _f32)
        return carry
    lax.fori_loop(0, NP // 16, zero, 0)

    def body(j, carry):
        rv = rowb[pl.ds(j * 16, 16)]
        cv = colb[pl.ds(j * 16, 16)]
        sr = plsc.load_gather(seltab, [rv])
        sc_ = plsc.load_gather(seltab, [cv])
        valid = (sr > 0) & (sc_ > 0)
        newc = jnp.where(valid, cv, jnp.full((16,), DUMMY, _i32))
        colob[pl.ds(j * 16, 16)] = newc
        plsc.addupdate_scatter(ldeg, [newc], jnp.full((16,), 1.0, _f32))
        return carry
    lax.fori_loop(0, EDGES_TILE // 16, body, 0)
    pltpu.sync_copy(colob, colout.at[pl.ds(w * EDGES_TILE, EDGES_TILE)])
    pltpu.sync_copy(ldeg, degout.at[w])


def _tca_body(x_ref, w_ref, deg32_ref, xw_ref, xws0_ref, xws1_ref,
              dinv_ref, invdeg_ref):
    xw = jnp.dot(x_ref[...], w_ref[...], preferred_element_type=_f32)
    deg = jnp.sum(deg32_ref[...], axis=0) + 1.0
    dinv = lax.rsqrt(deg)
    invdeg = 1.0 / deg
    xws = (xw.reshape(NR, 128, F) * dinv[:, :, None]).reshape(NP, F)
    xw_ref[...] = xw
    xws0_ref[...] = xws[:, :128]
    xws1_ref[...] = xws[:, 128:]
    dinv_ref[...] = dinv
    invdeg_ref[...] = invdeg


_tca = pl.pallas_call(
    _tca_body,
    out_shape=[
        jax.ShapeDtypeStruct((NP, F), _f32),
        jax.ShapeDtypeStruct((NP, 128), _f32),
        jax.ShapeDtypeStruct((NP, 128), _f32),
        jax.ShapeDtypeStruct((NR, 128), _f32),
        jax.ShapeDtypeStruct((NR, 128), _f32),
    ],
)


def _tcb_body(agg_ref, xw_ref, dinv_ref, invdeg_ref, b_ref, ws_ref,
              validf_ref, x3_ref, xs_ref, xss_ref):
    aggf = jnp.concatenate([agg_ref[0], agg_ref[1]], axis=1)
    dinv = dinv_ref[...]
    a3 = (aggf.reshape(NR, 128, F) * dinv[:, :, None]
          + xw_ref[...].reshape(NR, 128, F) * invdeg_ref[...][:, :, None])
    x3 = jax.nn.relu(a3 + b_ref[...].reshape(1, 1, F)) * validf_ref[...][:, :, None]
    xs = jnp.sum(x3 * ws_ref[...].reshape(1, 1, F), axis=2)
    x3_ref[...] = x3
    xs_ref[...] = xs
    xss_ref[...] = xs * dinv


_tcb = pl.pallas_call(
    _tcb_body,
    out_shape=[
        jax.ShapeDtypeStruct((NR, 128, F), _f32),
        jax.ShapeDtypeStruct((NR, 128), _f32),
        jax.ShapeDtypeStruct((NR, 128), _f32),
    ],
)


def _tcc_body(k_sel, sagg32_ref, xs_ref, dinv_ref, invdeg_ref, bs_ref,
              validf_ref, x3_ref, racc_ref, xn3_ref, selpad_ref,
              validout_ref, raccout_ref):
    sagg = jnp.sum(sagg32_ref[...], axis=0)
    xs = xs_ref[...]
    score = dinv_ref[...] * sagg + xs * invdeg_ref[...] + bs_ref[...]
    keym = jnp.where(validf_ref[...] > 0.5, score, _f32(-3.4e38))
    bits = lax.bitcast_convert_type(keym, _i32)
    keys = bits ^ ((bits >> 31) & _i32(0x7FFFFFFF))

    def bis(i, lh):
        lo, hi = lh
        mid = (lo >> 1) + (hi >> 1) + (lo & hi & 1)
        cnt = jnp.sum((keys > mid).astype(_i32))
        take = cnt >= k_sel
        return (jnp.where(take, mid, lo), jnp.where(take, hi, mid))

    lo, _ = lax.fori_loop(0, 33, bis, (_i32(-2147483647 - 1), _i32(2147483647)))
    sel = keys > lo
    sel_f = sel.astype(_f32)
    gain = jnp.tanh(score) * sel_f
    xn3 = x3_ref[...] * gain[:, :, None]
    xm = xn3 + (sel_f[:, :, None] - 1.0) * _f32(3.4e38)
    rmax = jnp.max(xm.reshape(NP, F), axis=0)[None, :]
    rmean = (jnp.sum(xn3.reshape(NP, F), axis=0) * (1.0 / k_sel))[None, :]
    raccout_ref[...] = racc_ref[...] + jnp.concatenate([rmax, rmean], axis=1)
    xn3_ref[...] = xn3
    selpad_ref[...] = sel.astype(_i32)
    validout_ref[...] = sel_f


def _make_tcc(k_sel):
    return pl.pallas_call(
        functools.partial(_tcc_body, k_sel),
        out_shape=[
            jax.ShapeDtypeStruct((NR, 128, F), _f32),
            jax.ShapeDtypeStruct((NR, 128), _i32),
            jax.ShapeDtypeStruct((NR, 128), _f32),
            jax.ShapeDtypeStruct((1, 2 * F), _f32),
        ],
    )


_tcc_by_k = {k: _make_tcc(k) for k in (K1, K2, K3)}


def kernel(edge_index, edge_weight, feat, W1, b1, Ws1, bs1, W2, b2, Ws2, bs2,
           W3, b3, Ws3, bs3):
    row = edge_index[0].astype(_i32)
    col = edge_index[1].astype(_i32)
    row1d = jnp.concatenate([row, jnp.zeros((EP - E,), _i32)])
    col1d = jnp.concatenate([col, jnp.full((EP - E,), DUMMY, _i32)])
    row2d = row1d.reshape(EB, 128)
    x = jnp.concatenate([feat, jnp.zeros((NP - N, F), _f32)], axis=0)
    validf = jnp.concatenate(
        [jnp.ones((N,), _f32), jnp.zeros((NP - N,), _f32)]).reshape(NR, 128)
    selpad = validf.reshape(NP).astype(_i32)
    zeros2d = jnp.zeros((BW, 128), _f32)
    row3d = row1d.reshape(16, NB_SUB, BW)
    racc = jnp.zeros((1, 2 * F), _f32)

    col1d, deg32 = _sc_edges(row1d, col1d, selpad)

    layers = [
        (W1, b1, Ws1, bs1, K1, False),
        (W2, b2, Ws2, bs2, K2, False),
        (W3, b3, Ws3, bs3, K3, True),
    ]
    for W, b, Ws, bs, k_sel, last in layers:
        xw, xws0, xws1, dinv, invdeg = _tca(x, W, deg32.reshape(32, NR, 128))
        agg = _sc_agg(xws0, xws1, row3d, col1d.reshape(16, NB_SUB, BW), zeros2d)
        x3, xs, xss = _tcb(agg, xw, dinv, invdeg, b.reshape(1, F),
                           Ws.reshape(1, F), validf)
        sagg32 = _sc_sagg(row1d, col1d, xss.reshape(NP))
        xn3, selpad2d, validf, racc = _tcc_by_k[k_sel](
            sagg32.reshape(32, NR, 128), xs, dinv, invdeg, bs.reshape(1, 1),
            validf, x3, racc)
        if not last:
            col1d, deg32 = _sc_edges(row1d, col1d, selpad2d.reshape(NP))
        x = xn3.reshape(NP, F)
    return racc


# speedup vs baseline: 1.9438x; 1.9438x over previous
import functools
import math

import jax
import jax.numpy as jnp
from jax import lax
from jax.experimental import pallas as pl
from jax.experimental.pallas import tpu as pltpu
from jax.experimental.pallas import tpu_sc as plsc

N = 10000
E = 160000
F = 256
NP = 10240
NR = NP // 128
EP = 163840
EB = EP // 128
DUMMY = 10008
K1 = math.ceil(0.5 * N)
K2 = math.ceil(0.5 * K1)
K3 = math.ceil(0.5 * K2)

BW = 64
NB_REG = EP // 32 // BW
CHK = 16
NB_TILE = EB // 32
ROWS_SUB = NP // 16

_mesh = plsc.VectorSubcoreMesh(core_axis_name="c", subcore_axis_name="s")
_f32 = jnp.float32
_i32 = jnp.int32


@functools.partial(
    pl.kernel,
    mesh=_mesh,
    compiler_params=pltpu.CompilerParams(needs_layout_passes=False),
    out_type=jax.ShapeDtypeStruct((2, NP, 128), _f32),
    scratch_types=[
        pltpu.VMEM((CHK, BW), _i32),
        pltpu.VMEM((CHK, BW), _i32),
        pltpu.VMEM((32, 16), _i32),
        pltpu.VMEM((BW, 128), _f32),
        pltpu.VMEM_SHARED((NP, 128), _f32),
        pltpu.SemaphoreType.DMA,
    ],
)
def _sc_agg(xws0, xws1, row3d, col3d, cnt2d, zeros2d, out,
            idxr, idxc, cntb, gbuf, acc, sem):
    c = lax.axis_index("c")
    s = lax.axis_index("s")
    pltpu.sync_copy(cnt2d, cntb)
    pltpu.sync_copy(zeros2d, gbuf)
    for j in range(ROWS_SUB // BW):
        pltpu.sync_copy(gbuf, acc.at[pl.ds(s * ROWS_SUB + j * BW, BW)])
    plsc.subcore_barrier()

    def run(table):
        for t in range(2):
            w = 2 * s + t
            cnt = cntb[w, pl.ds(0, 16)][0]
            nbat = (cnt + (BW - 1)) // BW

            def chunk(q, carry):
                pltpu.sync_copy(row3d.at[w].at[pl.ds(q * CHK, CHK)], idxr)
                pltpu.sync_copy(col3d.at[w].at[pl.ds(q * CHK, CHK)], idxc)
                m = jnp.minimum(CHK, nbat - q * CHK)

                def body(b, carry2):
                    pltpu.async_copy(table.at[idxr.at[b]], gbuf, sem).wait()
                    pltpu.sync_copy(gbuf, acc.at[idxc.at[b]], add=True)
                    return carry2
                lax.fori_loop(0, m, body, 0)
                return carry
            lax.fori_loop(0, (nbat + (CHK - 1)) // CHK, chunk, 0)

    @pl.when(c == 0)
    def _():
        run(xws0)

    @pl.when(c == 1)
    def _():
        run(xws1)

    plsc.subcore_barrier()
    for j in range(ROWS_SUB // 128):
        pltpu.sync_copy(
            acc.at[pl.ds(s * ROWS_SUB + j * 128, 128)],
            out.at[c].at[pl.ds(s * ROWS_SUB + j * 128, 128)],
        )


EDGES_TILE = EP // 32


@functools.partial(
    pl.kernel,
    mesh=_mesh,
    compiler_params=pltpu.CompilerParams(needs_layout_passes=False),
    out_type=jax.ShapeDtypeStruct((32, NP), _f32),
    scratch_types=[
        pltpu.VMEM((EDGES_TILE,), _i32),
        pltpu.VMEM((EDGES_TILE,), _i32),
        pltpu.VMEM((NP,), _f32),
        pltpu.VMEM((NP,), _f32),
    ],
)
def _sc_sagg(row1d, col1d, xss, out, rowb, colb, tab, lacc):
    c = lax.axis_index("c")
    s = lax.axis_index("s")
    w = s * 2 + c
    pltpu.sync_copy(xss, tab)
    pltpu.sync_copy(row1d.at[pl.ds(w * EDGES_TILE, EDGES_TILE)], rowb)
    pltpu.sync_copy(col1d.at[pl.ds(w * EDGES_TILE, EDGES_TILE)], colb)

    def zero(i, carry):
        lacc[pl.ds(i * 16, 16)] = jnp.zeros((16,), _f32)
        return carry
    lax.fori_loop(0, NP // 16, zero, 0)

    def body(j, carry):
        rv = rowb[pl.ds(j * 16, 16)]
        cv = colb[pl.ds(j * 16, 16)]
        vals = plsc.load_gather(tab, [rv])
        plsc.addupdate_scatter(lacc, [cv], vals)
        return carry
    lax.fori_loop(0, EDGES_TILE // 16, body, 0)
    pltpu.sync_copy(lacc, out.at[w])


@functools.partial(
    pl.kernel,
    mesh=_mesh,
    compiler_params=pltpu.CompilerParams(needs_layout_passes=False),
    out_type=[
        jax.ShapeDtypeStruct((EP,), _i32),
        jax.ShapeDtypeStruct((EP,), _i32),
        jax.ShapeDtypeStruct((32, NP), _f32),
        jax.ShapeDtypeStruct((32, 16), _i32),
    ],
    scratch_types=[
        pltpu.VMEM((EDGES_TILE,), _i32),
        pltpu.VMEM((EDGES_TILE,), _i32),
        pltpu.VMEM((EDGES_TILE,), _i32),
        pltpu.VMEM((EDGES_TILE,), _i32),
        pltpu.VMEM((16,), _i32),
        pltpu.VMEM((NP,), _i32),
        pltpu.VMEM((NP,), _f32),
    ],
)
def _sc_edges(row1d, col1d, selpad, colout, rowout, degout, cntout,
              rowb, colb, colob, rowob, cntb, seltab, ldeg):
    c = lax.axis_index("c")
    s = lax.axis_index("s")
    w = s * 2 + c
    pltpu.sync_copy(selpad, seltab)
    pltpu.sync_copy(row1d.at[pl.ds(w * EDGES_TILE, EDGES_TILE)], rowb)
    pltpu.sync_copy(col1d.at[pl.ds(w * EDGES_TILE, EDGES_TILE)], colb)

    def zero(i, carry):
        ldeg[pl.ds(i * 16, 16)] = jnp.zeros((16,), _f32)
        return carry
    lax.fori_loop(0, NP // 16, zero, 0)

    def prefill(i, carry):
        colob[pl.ds(i * 16, 16)] = jnp.full((16,), DUMMY, _i32)
        rowob[pl.ds(i * 16, 16)] = jnp.zeros((16,), _i32)
        return carry
    lax.fori_loop(0, EDGES_TILE // 16, prefill, 0)

    def body(j, off):
        rv = rowb[pl.ds(j * 16, 16)]
        cv = colb[pl.ds(j * 16, 16)]
        sr = plsc.load_gather(seltab, [rv])
        sc_ = plsc.load_gather(seltab, [cv])
        valid = (sr > 0) & (sc_ > 0)
        newc = jnp.where(valid, cv, jnp.full((16,), DUMMY, _i32))
        plsc.addupdate_scatter(ldeg, [newc], jnp.full((16,), 1.0, _f32))
        plsc.store_compressed(colob.at[pl.ds(off, 16)], cv, mask=valid)
        plsc.store_compressed(rowob.at[pl.ds(off, 16)], rv, mask=valid)
        return off + jnp.sum(valid.astype(_i32), axis=0)
    cnt = lax.fori_loop(0, EDGES_TILE // 16, body, _i32(0))
    cntb[pl.ds(0, 16)] = jnp.full((16,), 1, _i32) * cnt
    pltpu.sync_copy(colob, colout.at[pl.ds(w * EDGES_TILE, EDGES_TILE)])
    pltpu.sync_copy(rowob, rowout.at[pl.ds(w * EDGES_TILE, EDGES_TILE)])
    pltpu.sync_copy(cntb, cntout.at[w])
    pltpu.sync_copy(ldeg, degout.at[w])


def _tca_body(x_ref, w_ref, deg32_ref, xw_ref, xws0_ref, xws1_ref,
              dinv_ref, invdeg_ref):
    xw = jnp.dot(x_ref[...], w_ref[...], preferred_element_type=_f32)
    deg = jnp.sum(deg32_ref[...], axis=0) + 1.0
    dinv = lax.rsqrt(deg)
    invdeg = 1.0 / deg
    xws = (xw.reshape(NR, 128, F) * dinv[:, :, None]).reshape(NP, F)
    xw_ref[...] = xw
    xws0_ref[...] = xws[:, :128]
    xws1_ref[...] = xws[:, 128:]
    dinv_ref[...] = dinv
    invdeg_ref[...] = invdeg


_tca = pl.pallas_call(
    _tca_body,
    out_shape=[
        jax.ShapeDtypeStruct((NP, F), _f32),
        jax.ShapeDtypeStruct((NP, 128), _f32),
        jax.ShapeDtypeStruct((NP, 128), _f32),
        jax.ShapeDtypeStruct((NR, 128), _f32),
        jax.ShapeDtypeStruct((NR, 128), _f32),
    ],
)


def _tcb_body(agg_ref, xw_ref, dinv_ref, invdeg_ref, b_ref, ws_ref,
              validf_ref, x3_ref, xs_ref, xss_ref):
    aggf = jnp.concatenate([agg_ref[0], agg_ref[1]], axis=1)
    dinv = dinv_ref[...]
    a3 = (aggf.reshape(NR, 128, F) * dinv[:, :, None]
          + xw_ref[...].reshape(NR, 128, F) * invdeg_ref[...][:, :, None])
    x3 = jax.nn.relu(a3 + b_ref[...].reshape(1, 1, F)) * validf_ref[...][:, :, None]
    xs = jnp.sum(x3 * ws_ref[...].reshape(1, 1, F), axis=2)
    x3_ref[...] = x3
    xs_ref[...] = xs
    xss_ref[...] = xs * dinv


_tcb = pl.pallas_call(
    _tcb_body,
    out_shape=[
        jax.ShapeDtypeStruct((NR, 128, F), _f32),
        jax.ShapeDtypeStruct((NR, 128), _f32),
        jax.ShapeDtypeStruct((NR, 128), _f32),
    ],
)


def _tcc_body(k_sel, sagg32_ref, xs_ref, dinv_ref, invdeg_ref, bs_ref,
              validf_ref, x3_ref, racc_ref, xn3_ref, selpad_ref,
              validout_ref, raccout_ref):
    sagg = jnp.sum(sagg32_ref[...], axis=0)
    xs = xs_ref[...]
    score = dinv_ref[...] * sagg + xs * invdeg_ref[...] + bs_ref[...]
    keym = jnp.where(validf_ref[...] > 0.5, score, _f32(-3.4e38))
    bits = lax.bitcast_convert_type(keym, _i32)
    keys = bits ^ ((bits >> 31) & _i32(0x7FFFFFFF))

    def bis(i, lh):
        lo, hi = lh
        mid = (lo >> 1) + (hi >> 1) + (lo & hi & 1)
        cnt = jnp.sum((keys > mid).astype(_i32))
        take = cnt >= k_sel
        return (jnp.where(take, mid, lo), jnp.where(take, hi, mid))

    lo, _ = lax.fori_loop(0, 33, bis, (_i32(-2147483647 - 1), _i32(2147483647)))
    sel = keys > lo
    sel_f = sel.astype(_f32)
    gain = jnp.tanh(score) * sel_f
    xn3 = x3_ref[...] * gain[:, :, None]
    xm = xn3 + (sel_f[:, :, None] - 1.0) * _f32(3.4e38)
    rmax = jnp.max(xm.reshape(NP, F), axis=0)[None, :]
    rmean = (jnp.sum(xn3.reshape(NP, F), axis=0) * (1.0 / k_sel))[None, :]
    raccout_ref[...] = racc_ref[...] + jnp.concatenate([rmax, rmean], axis=1)
    xn3_ref[...] = xn3
    selpad_ref[...] = sel.astype(_i32)
    validout_ref[...] = sel_f


def _make_tcc(k_sel):
    return pl.pallas_call(
        functools.partial(_tcc_body, k_sel),
        out_shape=[
            jax.ShapeDtypeStruct((NR, 128, F), _f32),
            jax.ShapeDtypeStruct((NR, 128), _i32),
            jax.ShapeDtypeStruct((NR, 128), _f32),
            jax.ShapeDtypeStruct((1, 2 * F), _f32),
        ],
    )


_tcc_by_k = {k: _make_tcc(k) for k in (K1, K2, K3)}


def kernel(edge_index, edge_weight, feat, W1, b1, Ws1, bs1, W2, b2, Ws2, bs2,
           W3, b3, Ws3, bs3):
    row = edge_index[0].astype(_i32)
    col = edge_index[1].astype(_i32)
    row1d = jnp.concatenate([row, jnp.zeros((EP - E,), _i32)])
    col1d = jnp.concatenate([col, jnp.full((EP - E,), DUMMY, _i32)])
    x = jnp.concatenate([feat, jnp.zeros((NP - N, F), _f32)], axis=0)
    validf = jnp.concatenate(
        [jnp.ones((N,), _f32), jnp.zeros((NP - N,), _f32)]).reshape(NR, 128)
    selpad = validf.reshape(NP).astype(_i32)
    zeros2d = jnp.zeros((BW, 128), _f32)
    racc = jnp.zeros((1, 2 * F), _f32)

    col1d, rowc, deg32, cnt2d = _sc_edges(row1d, col1d, selpad)

    layers = [
        (W1, b1, Ws1, bs1, K1, False),
        (W2, b2, Ws2, bs2, K2, False),
        (W3, b3, Ws3, bs3, K3, True),
    ]
    for W, b, Ws, bs, k_sel, last in layers:
        xw, xws0, xws1, dinv, invdeg = _tca(x, W, deg32.reshape(32, NR, 128))
        agg = _sc_agg(xws0, xws1, rowc.reshape(32, NB_REG, BW),
                      col1d.reshape(32, NB_REG, BW), cnt2d, zeros2d)
        x3, xs, xss = _tcb(agg, xw, dinv, invdeg, b.reshape(1, F),
                           Ws.reshape(1, F), validf)
        sagg32 = _sc_sagg(rowc, col1d, xss.reshape(NP))
        xn3, selpad2d, validf, racc = _tcc_by_k[k_sel](
            sagg32.reshape(32, NR, 128), xs, dinv, invdeg, bs.reshape(1, 1),
            validf, x3, racc)
        if not last:
            col1d, rowc, deg32, cnt2d = _sc_edges(rowc, col1d, selpad2d.reshape(NP))
        x = xn3.reshape(NP, F)
    return racc


# speedup vs baseline: 1.9699x; 1.0135x over previous
import functools
import math

import jax
import jax.numpy as jnp
from jax import lax
from jax.experimental import pallas as pl
from jax.experimental.pallas import tpu as pltpu
from jax.experimental.pallas import tpu_sc as plsc

N = 10000
E = 160000
F = 256
NP = 10240
NR = NP // 128
EP = 163840
EB = EP // 128
DUMMY = 10008
K1 = math.ceil(0.5 * N)
K2 = math.ceil(0.5 * K1)
K3 = math.ceil(0.5 * K2)

BW = 128
NB_REG = EP // 32 // BW
CHK = 8
NB_TILE = EB // 32
ROWS_SUB = NP // 16

_mesh = plsc.VectorSubcoreMesh(core_axis_name="c", subcore_axis_name="s")
_f32 = jnp.float32
_i32 = jnp.int32


@functools.partial(
    pl.kernel,
    mesh=_mesh,
    compiler_params=pltpu.CompilerParams(needs_layout_passes=False),
    out_type=jax.ShapeDtypeStruct((2, NP, 128), _f32),
    scratch_types=[
        pltpu.VMEM((CHK, BW), _i32),
        pltpu.VMEM((CHK, BW), _i32),
        pltpu.VMEM((32, 16), _i32),
        pltpu.VMEM((BW, 128), _f32),
        pltpu.VMEM_SHARED((NP, 128), _f32),
        pltpu.SemaphoreType.DMA,
    ],
)
def _sc_agg(xws0, xws1, row3d, col3d, cnt2d, zeros2d, out,
            idxr, idxc, cntb, gbuf, acc, sem):
    c = lax.axis_index("c")
    s = lax.axis_index("s")
    pltpu.sync_copy(cnt2d, cntb)
    pltpu.sync_copy(zeros2d, gbuf)
    for j in range(ROWS_SUB // BW):
        pltpu.sync_copy(gbuf, acc.at[pl.ds(s * ROWS_SUB + j * BW, BW)])
    plsc.subcore_barrier()

    def run(table):
        for t in range(2):
            w = 2 * s + t
            cnt = cntb[w, pl.ds(0, 16)][0]
            nbat = (cnt + (BW - 1)) // BW

            def chunk(q, carry):
                pltpu.sync_copy(row3d.at[w].at[pl.ds(q * CHK, CHK)], idxr)
                pltpu.sync_copy(col3d.at[w].at[pl.ds(q * CHK, CHK)], idxc)
                m = jnp.minimum(CHK, nbat - q * CHK)

                def body(b, carry2):
                    pltpu.async_copy(table.at[idxr.at[b]], gbuf, sem).wait()
                    pltpu.sync_copy(gbuf, acc.at[idxc.at[b]], add=True)
                    return carry2
                lax.fori_loop(0, m, body, 0)
                return carry
            lax.fori_loop(0, (nbat + (CHK - 1)) // CHK, chunk, 0)

    @pl.when(c == 0)
    def _():
        run(xws0)

    @pl.when(c == 1)
    def _():
        run(xws1)

    plsc.subcore_barrier()
    for j in range(ROWS_SUB // 128):
        pltpu.sync_copy(
            acc.at[pl.ds(s * ROWS_SUB + j * 128, 128)],
            out.at[c].at[pl.ds(s * ROWS_SUB + j * 128, 128)],
        )


EDGES_TILE = EP // 32


@functools.partial(
    pl.kernel,
    mesh=_mesh,
    compiler_params=pltpu.CompilerParams(needs_layout_passes=False),
    out_type=jax.ShapeDtypeStruct((32, NP), _f32),
    scratch_types=[
        pltpu.VMEM((EDGES_TILE,), _i32),
        pltpu.VMEM((EDGES_TILE,), _i32),
        pltpu.VMEM((NP,), _f32),
        pltpu.VMEM((NP,), _f32),
    ],
)
def _sc_sagg(row1d, col1d, xss, out, rowb, colb, tab, lacc):
    c = lax.axis_index("c")
    s = lax.axis_index("s")
    w = s * 2 + c
    pltpu.sync_copy(xss, tab)
    pltpu.sync_copy(row1d.at[pl.ds(w * EDGES_TILE, EDGES_TILE)], rowb)
    pltpu.sync_copy(col1d.at[pl.ds(w * EDGES_TILE, EDGES_TILE)], colb)

    def zero(i, carry):
        lacc[pl.ds(i * 16, 16)] = jnp.zeros((16,), _f32)
        return carry
    lax.fori_loop(0, NP // 16, zero, 0)

    def body(j, carry):
        rv = rowb[pl.ds(j * 16, 16)]
        cv = colb[pl.ds(j * 16, 16)]
        vals = plsc.load_gather(tab, [rv])
        plsc.addupdate_scatter(lacc, [cv], vals)
        return carry
    lax.fori_loop(0, EDGES_TILE // 16, body, 0)
    pltpu.sync_copy(lacc, out.at[w])


@functools.partial(
    pl.kernel,
    mesh=_mesh,
    compiler_params=pltpu.CompilerParams(needs_layout_passes=False),
    out_type=[
        jax.ShapeDtypeStruct((EP,), _i32),
        jax.ShapeDtypeStruct((EP,), _i32),
        jax.ShapeDtypeStruct((32, NP), _f32),
        jax.ShapeDtypeStruct((32, 16), _i32),
    ],
    scratch_types=[
        pltpu.VMEM((EDGES_TILE,), _i32),
        pltpu.VMEM((EDGES_TILE,), _i32),
        pltpu.VMEM((EDGES_TILE,), _i32),
        pltpu.VMEM((EDGES_TILE,), _i32),
        pltpu.VMEM((16,), _i32),
        pltpu.VMEM((NP,), _i32),
        pltpu.VMEM((NP,), _f32),
    ],
)
def _sc_edges(row1d, col1d, selpad, colout, rowout, degout, cntout,
              rowb, colb, colob, rowob, cntb, seltab, ldeg):
    c = lax.axis_index("c")
    s = lax.axis_index("s")
    w = s * 2 + c
    pltpu.sync_copy(selpad, seltab)
    pltpu.sync_copy(row1d.at[pl.ds(w * EDGES_TILE, EDGES_TILE)], rowb)
    pltpu.sync_copy(col1d.at[pl.ds(w * EDGES_TILE, EDGES_TILE)], colb)

    def zero(i, carry):
        ldeg[pl.ds(i * 16, 16)] = jnp.zeros((16,), _f32)
        return carry
    lax.fori_loop(0, NP // 16, zero, 0)

    def prefill(i, carry):
        colob[pl.ds(i * 16, 16)] = jnp.full((16,), DUMMY, _i32)
        rowob[pl.ds(i * 16, 16)] = jnp.zeros((16,), _i32)
        return carry
    lax.fori_loop(0, EDGES_TILE // 16, prefill, 0)

    def body(j, off):
        rv = rowb[pl.ds(j * 16, 16)]
        cv = colb[pl.ds(j * 16, 16)]
        sr = plsc.load_gather(seltab, [rv])
        sc_ = plsc.load_gather(seltab, [cv])
        valid = (sr > 0) & (sc_ > 0)
        newc = jnp.where(valid, cv, jnp.full((16,), DUMMY, _i32))
        plsc.addupdate_scatter(ldeg, [newc], jnp.full((16,), 1.0, _f32))
        plsc.store_compressed(colob.at[pl.ds(off, 16)], cv, mask=valid)
        plsc.store_compressed(rowob.at[pl.ds(off, 16)], rv, mask=valid)
        return off + jnp.sum(valid.astype(_i32), axis=0)
    cnt = lax.fori_loop(0, EDGES_TILE // 16, body, _i32(0))
    cntb[pl.ds(0, 16)] = jnp.full((16,), 1, _i32) * cnt
    pltpu.sync_copy(colob, colout.at[pl.ds(w * EDGES_TILE, EDGES_TILE)])
    pltpu.sync_copy(rowob, rowout.at[pl.ds(w * EDGES_TILE, EDGES_TILE)])
    pltpu.sync_copy(cntb, cntout.at[w])
    pltpu.sync_copy(ldeg, degout.at[w])


def _tca_body(x_ref, w_ref, deg32_ref, xw_ref, xws0_ref, xws1_ref,
              dinv_ref, invdeg_ref):
    xw = jnp.dot(x_ref[...], w_ref[...], preferred_element_type=_f32)
    deg = jnp.sum(deg32_ref[...], axis=0) + 1.0
    dinv = lax.rsqrt(deg)
    invdeg = 1.0 / deg
    xws = (xw.reshape(NR, 128, F) * dinv[:, :, None]).reshape(NP, F)
    xw_ref[...] = xw
    xws0_ref[...] = xws[:, :128]
    xws1_ref[...] = xws[:, 128:]
    dinv_ref[...] = dinv
    invdeg_ref[...] = invdeg


_tca = pl.pallas_call(
    _tca_body,
    out_shape=[
        jax.ShapeDtypeStruct((NP, F), _f32),
        jax.ShapeDtypeStruct((NP, 128), _f32),
        jax.ShapeDtypeStruct((NP, 128), _f32),
        jax.ShapeDtypeStruct((NR, 128), _f32),
        jax.ShapeDtypeStruct((NR, 128), _f32),
    ],
)


def _tcb_body(agg_ref, xw_ref, dinv_ref, invdeg_ref, b_ref, ws_ref,
              validf_ref, x3_ref, xs_ref, xss_ref):
    aggf = jnp.concatenate([agg_ref[0], agg_ref[1]], axis=1)
    dinv = dinv_ref[...]
    a3 = (aggf.reshape(NR, 128, F) * dinv[:, :, None]
          + xw_ref[...].reshape(NR, 128, F) * invdeg_ref[...][:, :, None])
    x3 = jax.nn.relu(a3 + b_ref[...].reshape(1, 1, F)) * validf_ref[...][:, :, None]
    xs = jnp.sum(x3 * ws_ref[...].reshape(1, 1, F), axis=2)
    x3_ref[...] = x3
    xs_ref[...] = xs
    xss_ref[...] = xs * dinv


_tcb = pl.pallas_call(
    _tcb_body,
    out_shape=[
        jax.ShapeDtypeStruct((NR, 128, F), _f32),
        jax.ShapeDtypeStruct((NR, 128), _f32),
        jax.ShapeDtypeStruct((NR, 128), _f32),
    ],
)


def _tcc_body(k_sel, sagg32_ref, xs_ref, dinv_ref, invdeg_ref, bs_ref,
              validf_ref, x3_ref, racc_ref, xn3_ref, selpad_ref,
              validout_ref, raccout_ref):
    sagg = jnp.sum(sagg32_ref[...], axis=0)
    xs = xs_ref[...]
    score = dinv_ref[...] * sagg + xs * invdeg_ref[...] + bs_ref[...]
    keym = jnp.where(validf_ref[...] > 0.5, score, _f32(-3.4e38))
    bits = lax.bitcast_convert_type(keym, _i32)
    keys = bits ^ ((bits >> 31) & _i32(0x7FFFFFFF))

    def bis(i, lh):
        lo, hi = lh
        mid = (lo >> 1) + (hi >> 1) + (lo & hi & 1)
        cnt = jnp.sum((keys > mid).astype(_i32))
        take = cnt >= k_sel
        return (jnp.where(take, mid, lo), jnp.where(take, hi, mid))

    lo, _ = lax.fori_loop(0, 33, bis, (_i32(-2147483647 - 1), _i32(2147483647)))
    sel = keys > lo
    sel_f = sel.astype(_f32)
    gain = jnp.tanh(score) * sel_f
    xn3 = x3_ref[...] * gain[:, :, None]
    xm = xn3 + (sel_f[:, :, None] - 1.0) * _f32(3.4e38)
    rmax = jnp.max(xm.reshape(NP, F), axis=0)[None, :]
    rmean = (jnp.sum(xn3.reshape(NP, F), axis=0) * (1.0 / k_sel))[None, :]
    raccout_ref[...] = racc_ref[...] + jnp.concatenate([rmax, rmean], axis=1)
    xn3_ref[...] = xn3
    selpad_ref[...] = sel.astype(_i32)
    validout_ref[...] = sel_f


def _make_tcc(k_sel):
    return pl.pallas_call(
        functools.partial(_tcc_body, k_sel),
        out_shape=[
            jax.ShapeDtypeStruct((NR, 128, F), _f32),
            jax.ShapeDtypeStruct((NR, 128), _i32),
            jax.ShapeDtypeStruct((NR, 128), _f32),
            jax.ShapeDtypeStruct((1, 2 * F), _f32),
        ],
    )


_tcc_by_k = {k: _make_tcc(k) for k in (K1, K2, K3)}


def kernel(edge_index, edge_weight, feat, W1, b1, Ws1, bs1, W2, b2, Ws2, bs2,
           W3, b3, Ws3, bs3):
    row = edge_index[0].astype(_i32)
    col = edge_index[1].astype(_i32)
    row1d = jnp.concatenate([row, jnp.zeros((EP - E,), _i32)])
    col1d = jnp.concatenate([col, jnp.full((EP - E,), DUMMY, _i32)])
    x = jnp.concatenate([feat, jnp.zeros((NP - N, F), _f32)], axis=0)
    validf = jnp.concatenate(
        [jnp.ones((N,), _f32), jnp.zeros((NP - N,), _f32)]).reshape(NR, 128)
    selpad = validf.reshape(NP).astype(_i32)
    zeros2d = jnp.zeros((BW, 128), _f32)
    racc = jnp.zeros((1, 2 * F), _f32)

    col1d, rowc, deg32, cnt2d = _sc_edges(row1d, col1d, selpad)

    layers = [
        (W1, b1, Ws1, bs1, K1, False),
        (W2, b2, Ws2, bs2, K2, False),
        (W3, b3, Ws3, bs3, K3, True),
    ]
    for W, b, Ws, bs, k_sel, last in layers:
        xw, xws0, xws1, dinv, invdeg = _tca(x, W, deg32.reshape(32, NR, 128))
        agg = _sc_agg(xws0, xws1, rowc.reshape(32, NB_REG, BW),
                      col1d.reshape(32, NB_REG, BW), cnt2d, zeros2d)
        x3, xs, xss = _tcb(agg, xw, dinv, invdeg, b.reshape(1, F),
                           Ws.reshape(1, F), validf)
        sagg32 = _sc_sagg(rowc, col1d, xss.reshape(NP))
        xn3, selpad2d, validf, racc = _tcc_by_k[k_sel](
            sagg32.reshape(32, NR, 128), xs, dinv, invdeg, bs.reshape(1, 1),
            validf, x3, racc)
        if not last:
            col1d, rowc, deg32, cnt2d = _sc_edges(rowc, col1d, selpad2d.reshape(NP))
        x = xn3.reshape(NP, F)
    return racc
